# Initial kernel scaffold; baseline (speedup 1.0000x reference)
import jax
import jax.numpy as jnp
from jax.experimental import pallas as pl


def kernel(faces, vertices, face_vertices, angles, face_areas, normals, edge_list, pad_value, emb_v, emb_a, emb_ar, emb_n, W_fn, b_fn, W_es, W_en, b_enc, codebook, W_d1, b_d1, W_d2, b_d2):
    raise NotImplementedError("write your pallas kernel here")


# speedup vs baseline: 1.5133x; 1.5133x over previous
import math
import functools

import jax
import jax.numpy as jnp
from jax import lax
from jax.experimental import pallas as pl

F_FACES = 10000
N_VERTS = 5000
N_EDGES = 30000
NUMD = 128
VQ_DIM = 192
CB_N = 16384
D_GRAPH = 196
D_ENC = 576

_LOG1E4_64 = math.log(10000.0) / 64.0


def _gauss_kernel5():
    s = 0.4
    vals = [math.exp(-(x * x) / (2.0 * s * s)) for x in (-2.0, -1.0, 0.0, 1.0, 2.0)]
    tot = sum(vals)
    return [v / tot for v in vals]


_GK5 = _gauss_kernel5()


def _embed_body(fv_ref, ang_ref, ar_ref, nrm_ref,
                emb_v_ref, emb_a_ref, emb_ar_ref, emb_n_ref,
                w_ref, b_ref, graph_ref, vd_ref, *, fb):
    fv = fv_ref[...]
    ang = ang_ref[...]
    ar = ar_ref[...]
    nrm = nrm_ref[...]

    vd = jnp.round((fv + 1.0) * 0.5 * (NUMD - 1)).astype(jnp.int32)
    ad = jnp.round(ang / math.pi * (NUMD - 1)).astype(jnp.int32)
    ard = jnp.round(ar / 4.0 * (NUMD - 1)).astype(jnp.int32)
    nd = jnp.round((nrm + 1.0) * 0.5 * (NUMD - 1)).astype(jnp.int32)

    lanes = lax.broadcasted_iota(jnp.int32, (fb, NUMD), 1)
    w = w_ref[...]
    emb_v = emb_v_ref[...]
    emb_a = emb_a_ref[...]
    emb_ar = emb_ar_ref[...]
    emb_n = emb_n_ref[...]

    acc = jnp.zeros((fb, D_GRAPH), jnp.float32)
    wpe = jnp.zeros((64, D_GRAPH), jnp.float32)
    for j in range(9):
        ws = w[j * 64:(j + 1) * 64, :]
        wpe = wpe + ws
        oh = (lanes == vd[:, j:j + 1]).astype(jnp.float32)
        m = jnp.dot(emb_v, ws, preferred_element_type=jnp.float32)
        acc = acc + jnp.dot(oh, m, preferred_element_type=jnp.float32)

    pid = pl.program_id(0)
    pos = (pid * fb + lax.broadcasted_iota(jnp.int32, (fb, 1), 0)).astype(jnp.float32)
    d_i = lax.broadcasted_iota(jnp.int32, (1, 64), 1)
    d_even = (d_i - (d_i % 2)).astype(jnp.float32)
    freq = jnp.exp(-d_even * _LOG1E4_64)
    ph = pos * freq
    pe = jnp.where((d_i % 2) == 0, jnp.sin(ph), jnp.cos(ph))
    acc = acc + jnp.dot(pe, wpe, preferred_element_type=jnp.float32)

    for j in range(3):
        ws = w[576 + j * 64:576 + (j + 1) * 64, :]
        oh = (lanes == nd[:, j:j + 1]).astype(jnp.float32)
        m = jnp.dot(emb_n, ws, preferred_element_type=jnp.float32)
        acc = acc + jnp.dot(oh, m, preferred_element_type=jnp.float32)

    for j in range(3):
        ws = w[768 + j * 16:768 + (j + 1) * 16, :]
        oh = (lanes == ad[:, j:j + 1]).astype(jnp.float32)
        m = jnp.dot(emb_a, ws, preferred_element_type=jnp.float32)
        acc = acc + jnp.dot(oh, m, preferred_element_type=jnp.float32)

    ws = w[816:832, :]
    oh = (lanes == ard).astype(jnp.float32)
    m = jnp.dot(emb_ar, ws, preferred_element_type=jnp.float32)
    acc = acc + jnp.dot(oh, m, preferred_element_type=jnp.float32)

    graph_ref[...] = acc + b_ref[...]
    vd_ref[...] = vd


def _embed_stage(fv, ang, ar, nrm, emb_v, emb_a, emb_ar, emb_n, w_fn, b_fn):
    fb = 2000
    grid = (F_FACES // fb,)
    full = lambda shape: pl.BlockSpec(shape, lambda i: (0,) * len(shape))
    row = lambda cols: pl.BlockSpec((fb, cols), lambda i: (i, 0))
    graph, vd = pl.pallas_call(
        functools.partial(_embed_body, fb=fb),
        grid=grid,
        in_specs=[row(9), row(3), row(1), row(3),
                  full((NUMD, 64)), full((NUMD, 16)), full((NUMD, 16)),
                  full((NUMD, 64)), full((832, D_GRAPH)), full((1, D_GRAPH))],
        out_specs=[row(D_GRAPH), row(9)],
        out_shape=[jax.ShapeDtypeStruct((F_FACES, D_GRAPH), jnp.float32),
                   jax.ShapeDtypeStruct((F_FACES, 9), jnp.int32)],
    )(fv, ang, ar, nrm, emb_v, emb_a, emb_ar, emb_n, w_fn, b_fn)
    return graph, vd


def _gather_body(idx_ref, tab_ref, out_ref, *, mb, nb):
    ni = pl.program_id(1)
    idx = idx_ref[0, 0, :]
    loc = idx[:, None] - ni * nb
    oh = (loc == lax.broadcasted_iota(jnp.int32, (mb, nb), 1)).astype(jnp.float32)
    part = jnp.dot(oh, tab_ref[...], preferred_element_type=jnp.float32)

    @pl.when(ni == 0)
    def _():
        out_ref[...] = part

    @pl.when(ni > 0)
    def _():
        out_ref[...] = out_ref[...] + part


def _gather_rows(tab, idx, mb, nb):
    m = idx.shape[0]
    n, d = tab.shape
    idx3 = idx.reshape(m // mb, 1, mb)
    return pl.pallas_call(
        functools.partial(_gather_body, mb=mb, nb=nb),
        grid=(m // mb, n // nb),
        in_specs=[pl.BlockSpec((1, 1, mb), lambda mi, ni: (mi, 0, 0)),
                  pl.BlockSpec((nb, d), lambda mi, ni: (ni, 0))],
        out_specs=pl.BlockSpec((mb, d), lambda mi, ni: (mi, 0)),
        out_shape=jax.ShapeDtypeStruct((m, d), jnp.float32),
    )(idx3, tab)


def _scatter_body(idx_ref, rows_ref, out_ref, cnt_ref, *, mb, nb, n_mi, mean):
    ni = pl.program_id(0)
    mi = pl.program_id(1)
    idx = idx_ref[0, 0, :]
    loc = lax.broadcasted_iota(jnp.int32, (nb, mb), 0) + ni * nb
    oh = (loc == idx[None, :]).astype(jnp.float32)
    part = jnp.dot(oh, rows_ref[...], preferred_element_type=jnp.float32)
    cpart = jnp.sum(oh, axis=1, keepdims=True)

    @pl.when(mi == 0)
    def _():
        out_ref[...] = part
        cnt_ref[...] = cpart

    @pl.when(mi > 0)
    def _():
        out_ref[...] = out_ref[...] + part
        cnt_ref[...] = cnt_ref[...] + cpart

    if mean:
        @pl.when(mi == n_mi - 1)
        def _():
            out_ref[...] = out_ref[...] / jnp.maximum(cnt_ref[...], 1.0)


def _scatter_add_rows(rows, idx, n, mb, nb, mean=False):
    m, d = rows.shape
    idx3 = idx.reshape(m // mb, 1, mb)
    out, cnt = pl.pallas_call(
        functools.partial(_scatter_body, mb=mb, nb=nb, n_mi=m // mb, mean=mean),
        grid=(n // nb, m // mb),
        in_specs=[pl.BlockSpec((1, 1, mb), lambda ni, mi: (mi, 0, 0)),
                  pl.BlockSpec((mb, d), lambda ni, mi: (mi, 0))],
        out_specs=[pl.BlockSpec((nb, d), lambda ni, mi: (ni, 0)),
                   pl.BlockSpec((nb, 1), lambda ni, mi: (ni, 0))],
        out_shape=[jax.ShapeDtypeStruct((n, d), jnp.float32),
                   jax.ShapeDtypeStruct((n, 1), jnp.float32)],
    )(idx3, rows)
    return out, cnt


def _enc_body(g_ref, a_ref, wes_ref, wen_ref, b_ref, out_ref):
    x = jnp.dot(g_ref[...], wes_ref[...], preferred_element_type=jnp.float32)
    y = jnp.dot(a_ref[...], wen_ref[...], preferred_element_type=jnp.float32)
    out_ref[...] = jnp.maximum(x + y + b_ref[...], 0.0)


def _enc_stage(graph, agg, w_es, w_en, b_enc):
    fb = 2000
    return pl.pallas_call(
        _enc_body,
        grid=(F_FACES // fb,),
        in_specs=[pl.BlockSpec((fb, D_GRAPH), lambda i: (i, 0)),
                  pl.BlockSpec((fb, D_GRAPH), lambda i: (i, 0)),
                  pl.BlockSpec((D_GRAPH, D_ENC), lambda i: (0, 0)),
                  pl.BlockSpec((D_GRAPH, D_ENC), lambda i: (0, 0)),
                  pl.BlockSpec((1, D_ENC), lambda i: (0, 0))],
        out_specs=pl.BlockSpec((fb, D_ENC), lambda i: (i, 0)),
        out_shape=jax.ShapeDtypeStruct((F_FACES, D_ENC), jnp.float32),
    )(graph, agg, w_es, w_en, b_enc)


def _vq_body(r_ref, cbt_ref, idx_ref, msum_ref, best_ref, bidx_ref,
             *, vb, chunk, n_ci):
    ci = pl.program_id(1)
    r = r_ref[...]
    ct = cbt_ref[...]
    cb2 = jnp.sum(ct * ct, axis=0)[None, :]
    scores = cb2 - 2.0 * jnp.dot(r, ct, preferred_element_type=jnp.float32)
    lmin = jnp.min(scores, axis=1, keepdims=True)
    li = lax.broadcasted_iota(jnp.int32, (vb, chunk), 1) + ci * chunk
    larg = jnp.min(jnp.where(scores <= lmin, li, jnp.int32(2 ** 30)),
                   axis=1, keepdims=True)

    @pl.when(ci == 0)
    def _():
        best_ref[...] = lmin
        bidx_ref[...] = larg

    @pl.when(ci > 0)
    def _():
        upd = lmin < best_ref[...]
        best_ref[...] = jnp.where(upd, lmin, best_ref[...])
        bidx_ref[...] = jnp.where(upd, larg, bidx_ref[...])

    @pl.when(ci == n_ci - 1)
    def _():
        idx_ref[...] = bidx_ref[...]
        r2 = jnp.sum(r * r, axis=1, keepdims=True)
        msum_ref[...] = jnp.full((1, 1, 128), jnp.sum(best_ref[...] + r2),
                                 jnp.float32)


def _vq_stage(resid, codebook_t):
    from jax.experimental.pallas import tpu as pltpu
    vb, chunk = 1000, 512
    idx, msum = pl.pallas_call(
        functools.partial(_vq_body, vb=vb, chunk=chunk, n_ci=CB_N // chunk),
        grid=(N_VERTS // vb, CB_N // chunk),
        in_specs=[pl.BlockSpec((vb, VQ_DIM), lambda vi, ci: (vi, 0)),
                  pl.BlockSpec((VQ_DIM, chunk), lambda vi, ci: (0, ci))],
        out_specs=[pl.BlockSpec((vb, 1), lambda vi, ci: (vi, 0)),
                   pl.BlockSpec((1, 1, 128), lambda vi, ci: (vi, 0, 0))],
        out_shape=[jax.ShapeDtypeStruct((N_VERTS, 1), jnp.int32),
                   jax.ShapeDtypeStruct((N_VERTS // vb, 1, 128), jnp.float32)],
        scratch_shapes=[pltpu.VMEM((vb, 1), jnp.float32),
                        pltpu.VMEM((vb, 1), jnp.int32)],
    )(resid, codebook_t)
    return idx.reshape(N_VERTS), msum[:, 0, 0]


def _dec_body(qf_ref, vd_ref, w1_ref, b1_ref, w2_ref, b2_ref, out_ref, *, fb):
    h = jnp.maximum(
        jnp.dot(qf_ref[...], w1_ref[...], preferred_element_type=jnp.float32)
        + b1_ref[...], 0.0)
    dec = jnp.dot(h, w2_ref[...], preferred_element_type=jnp.float32) + b2_ref[...]

    rr = lax.broadcasted_iota(jnp.int32, (NUMD, NUMD), 0)
    cc = lax.broadcasted_iota(jnp.int32, (NUMD, NUMD), 1)
    band = jnp.zeros((NUMD, NUMD), jnp.float32)
    for k in range(5):
        band = band + _GK5[k] * ((cc - rr) == (k - 2)).astype(jnp.float32)

    vd = vd_ref[...]
    lanes = lax.broadcasted_iota(jnp.int32, (fb, NUMD), 1)
    tot = jnp.zeros((), jnp.float32)
    for j in range(9):
        sl = dec[:, j * NUMD:(j + 1) * NUMD]
        mx = jnp.max(sl, axis=1, keepdims=True)
        ls = sl - mx - jnp.log(jnp.sum(jnp.exp(sl - mx), axis=1, keepdims=True))
        s = jnp.dot(ls, band, preferred_element_type=jnp.float32)
        pick = jnp.where(lanes == vd[:, j:j + 1], s, 0.0)
        tot = tot + jnp.sum(pick)
    out_ref[...] = jnp.full((1, 1, 128), tot, jnp.float32)


def _dec_stage(qf, vd, w1, b1, w2, b2):
    fb = 1000
    parts = pl.pallas_call(
        functools.partial(_dec_body, fb=fb),
        grid=(F_FACES // fb,),
        in_specs=[pl.BlockSpec((fb, D_ENC), lambda i: (i, 0)),
                  pl.BlockSpec((fb, 9), lambda i: (i, 0)),
                  pl.BlockSpec((D_ENC, 512), lambda i: (0, 0)),
                  pl.BlockSpec((1, 512), lambda i: (0, 0)),
                  pl.BlockSpec((512, 1152), lambda i: (0, 0)),
                  pl.BlockSpec((1, 1152), lambda i: (0, 0))],
        out_specs=pl.BlockSpec((1, 1, 128), lambda i: (i, 0, 0)),
        out_shape=jax.ShapeDtypeStruct((F_FACES // fb, 1, 128), jnp.float32),
    )(qf, vd, w1, b1, w2, b2)
    return parts[:, 0, 0]


def kernel(faces, vertices, face_vertices, angles, face_areas, normals,
           edge_list, pad_value, emb_v, emb_a, emb_ar, emb_n, W_fn, b_fn,
           W_es, W_en, b_enc, codebook, W_d1, b_d1, W_d2, b_d2):
    fv = face_vertices[0]
    ang = angles[0]
    ar = face_areas[0].reshape(F_FACES, 1)
    nrm = normals[0]
    src = edge_list[0, 0].astype(jnp.int32)
    dst = edge_list[0, 1].astype(jnp.int32)
    faces = faces.astype(jnp.int32)

    graph, vd = _embed_stage(
        fv, ang, ar, nrm,
        emb_v.astype(jnp.float32), emb_a.astype(jnp.float32),
        emb_ar.astype(jnp.float32), emb_n.astype(jnp.float32),
        W_fn.astype(jnp.float32), b_fn.astype(jnp.float32).reshape(1, D_GRAPH))

    gs = _gather_rows(graph, src, mb=1000, nb=2000)
    agg, _ = _scatter_add_rows(gs, dst, F_FACES, mb=1000, nb=2000)

    enc = _enc_stage(graph, agg,
                     W_es.astype(jnp.float32), W_en.astype(jnp.float32),
                     b_enc.astype(jnp.float32).reshape(1, D_ENC))

    encv = enc.reshape(F_FACES * 3, VQ_DIM)
    vidx = faces.reshape(F_FACES * 3)
    vf, _ = _scatter_add_rows(encv, vidx, N_VERTS, mb=1000, nb=1000, mean=True)

    cbf = codebook.astype(jnp.float32)
    cbt = cbf.T
    idx1, m1 = _vq_stage(vf, cbt)
    qv1 = _gather_rows(cbf, idx1, mb=1000, nb=2048)
    resid2 = vf - qv1
    idx2, m2 = _vq_stage(resid2, cbt)
    qv2 = _gather_rows(cbf, idx2, mb=1000, nb=2048)
    quantized = qv1 + qv2
    commit = (jnp.sum(m1) + jnp.sum(m2)) / (N_VERTS * VQ_DIM)

    qf = _gather_rows(quantized, vidx, mb=1000, nb=1000).reshape(F_FACES, D_ENC)
    parts = _dec_stage(qf, vd,
                       W_d1.astype(jnp.float32),
                       b_d1.astype(jnp.float32).reshape(1, 512),
                       W_d2.astype(jnp.float32),
                       b_d2.astype(jnp.float32).reshape(1, 1152))
    recon = -jnp.sum(parts)
    return recon + commit
